# baseline (speedup 1.0000x reference)
import jax
import jax.numpy as jnp
from jax import lax
from jax.experimental import pallas as pl
from jax.experimental.pallas import tpu as pltpu
from jax.experimental.pallas import tpu_sc as plsc

NUM_EXPERTS = 8
TOP_K = 2
T = 2048
D = 768
FF = 2048

RT_TILE = 256
N_RT = T // RT_TILE

BLK = 256
NBLK = 24
NPAD = NBLK * BLK

NW = 32


def _route_body(x_ref, wmix_ref, wgate_ref,
                h_ref, logits_ref, probs_ref, zsum_ref, lbl_ref, load_ref,
                cnt_ref, blkexp_ref, pos01_ref, w01_ref,
                cnt1_scr, cnt2_scr, off_scr, m1_scr, m2_scr):
    i = pl.program_id(0)

    @pl.when(i < N_RT)
    def _sweep1():
        x = x_ref[...]
        xm = jnp.dot(x, wmix_ref[...], preferred_element_type=jnp.float32)
        hh = x + jnp.tanh(xm)
        h_ref[...] = hh.astype(jnp.bfloat16)
        logits = jnp.dot(hh, wgate_ref[...], preferred_element_type=jnp.float32)
        logits_ref[...] = logits
        mx = jnp.max(logits, axis=1, keepdims=True)
        ex = jnp.exp(logits - mx)
        den = jnp.sum(ex, axis=1, keepdims=True)
        probs = ex / den
        probs_ref[...] = probs

        e_iota = lax.broadcasted_iota(jnp.int32, probs.shape, 1)
        p1 = jnp.max(probs, axis=1, keepdims=True)
        is1 = probs == p1
        i1 = jnp.min(jnp.where(is1, e_iota, NUM_EXPERTS), axis=1, keepdims=True)
        sel1 = e_iota == i1
        pm = jnp.where(sel1, -jnp.inf, probs)
        p2 = jnp.max(pm, axis=1, keepdims=True)
        is2 = pm == p2
        i2 = jnp.min(jnp.where(is2, e_iota, NUM_EXPERTS), axis=1, keepdims=True)
        sel2 = e_iota == i2
        m1 = sel1.astype(jnp.float32)
        m2 = sel2.astype(jnp.float32)
        m1_scr[pl.ds(i * RT_TILE, RT_TILE), :] = m1
        m2_scr[pl.ds(i * RT_TILE, RT_TILE), :] = m2
        csum = jnp.sum(m1 + m2, axis=0, keepdims=True)
        wsum = p1 + p2
        w01_ref[...] = jnp.concatenate(
            [(p1 / wsum).reshape(1, RT_TILE), (p2 / wsum).reshape(1, RT_TILE)], axis=0)

        z = jnp.log(den[:, 0]) + mx[:, 0]
        z2 = jnp.sum(z * z).reshape(1, 1)
        psum = jnp.sum(probs, axis=0, keepdims=True)

        @pl.when(i == 0)
        def _init():
            zsum_ref[...] = z2
            load_ref[...] = psum
            cnt1_scr[...] = csum

        @pl.when(i > 0)
        def _acc():
            zsum_ref[...] += z2
            load_ref[...] += psum
            cnt1_scr[...] += csum

        @pl.when(i == N_RT - 1)
        def _fin():
            cnt = cnt1_scr[...]
            cnt_ref[...] = cnt.astype(jnp.int32)
            zsum_ref[...] = zsum_ref[...] / T
            load = load_ref[...] / T
            load_ref[...] = load
            frac = cnt / (T * TOP_K)
            lbl_ref[...] = (NUM_EXPERTS * jnp.sum(frac * load)).reshape(1, 1)
            nblk = jnp.floor((cnt + (BLK - 1)) * (1.0 / BLK))
            a = lax.broadcasted_iota(jnp.int32, (NUM_EXPERTS, NUM_EXPERTS), 0)
            b = lax.broadcasted_iota(jnp.int32, (NUM_EXPERTS, NUM_EXPERTS), 1)
            excl = (a < b).astype(jnp.float32)
            off = jnp.dot(nblk, excl, preferred_element_type=jnp.float32) * BLK
            off_scr[...] = off
            cb_end = ((off + nblk * BLK) * (1.0 / BLK)).astype(jnp.int32)
            bi = lax.broadcasted_iota(jnp.int32, (NBLK, NUM_EXPERTS), 0)
            ge = (bi >= jnp.broadcast_to(cb_end, (NBLK, NUM_EXPERTS))).astype(jnp.int32)
            be = jnp.sum(ge, axis=1).reshape(1, NBLK)
            blkexp_ref[...] = jnp.minimum(be, NUM_EXPERTS - 1)

    @pl.when(i == N_RT)
    def _init2():
        cnt2_scr[...] = jnp.zeros((1, NUM_EXPERTS), jnp.float32)

    @pl.when(i >= N_RT)
    def _sweep2():
        j = i - N_RT
        m1 = m1_scr[pl.ds(j * RT_TILE, RT_TILE), :]
        m2 = m2_scr[pl.ds(j * RT_TILE, RT_TILE), :]
        self_f = m1 + m2
        r = lax.broadcasted_iota(jnp.int32, (RT_TILE, RT_TILE), 0)
        c = lax.broadcasted_iota(jnp.int32, (RT_TILE, RT_TILE), 1)
        ltri = (c < r).astype(jnp.float32)
        rank_in = jnp.dot(ltri, self_f, preferred_element_type=jnp.float32)
        slot = off_scr[...] + cnt2_scr[...] + rank_in
        pos0 = jnp.sum(m1 * slot, axis=1)
        pos1 = jnp.sum(m2 * slot, axis=1)
        pos01_ref[...] = jnp.concatenate(
            [pos0.reshape(1, RT_TILE), pos1.reshape(1, RT_TILE)],
            axis=0).astype(jnp.int32)
        cnt2_scr[...] += jnp.sum(self_f, axis=0, keepdims=True)


def _sc_scatter_body(pos01_hbm, w01_hbm, ids_hbm, wq_hbm, pos_v, w_v, ids_v, wq_v):
    wid = lax.axis_index("s") * 2 + lax.axis_index("c")
    pltpu.sync_copy(pos01_hbm.at[0, pl.ds(0, T)], pos_v.at[pl.ds(0, T)])
    pltpu.sync_copy(pos01_hbm.at[1, pl.ds(0, T)], pos_v.at[pl.ds(T, T)])
    pltpu.sync_copy(w01_hbm.at[0, pl.ds(0, T)], w_v.at[pl.ds(0, T)])
    pltpu.sync_copy(w01_hbm.at[1, pl.ds(0, T)], w_v.at[pl.ds(T, T)])
    zeros_i = jnp.zeros((16,), jnp.int32)
    zeros_f = jnp.zeros((16,), jnp.float32)

    def _zero(j, _):
        ids_v[pl.ds(j * 16, 16)] = zeros_i
        wq_v[pl.ds(j * 16, 16)] = zeros_f
        return 0
    lax.fori_loop(0, NPAD // 16, _zero, 0)

    lane = lax.iota(jnp.int32, 16)

    def _scat(k, _):
        idx = pos_v[pl.ds(k * 16, 16)]
        tok = lax.rem(k * 16 + lane, T)
        plsc.store_scatter(ids_v, [idx], tok)
        vals = w_v[pl.ds(k * 16, 16)]
        plsc.store_scatter(wq_v, [idx], vals)
        return 0
    lax.fori_loop(0, (T * TOP_K) // 16, _scat, 0)

    span = NPAD // NW
    base = wid * span
    pltpu.sync_copy(ids_v.at[pl.ds(base, span)], ids_hbm.at[pl.ds(base, span)])
    pltpu.sync_copy(wq_v.at[pl.ds(base, span)], wq_hbm.at[pl.ds(base, span)])


def _ffn_body(blkexp_ref, ids_ref, h_ref, w1_ref, b1_ref, w2_ref, b2_ref,
              wq_ref, fin_ref):
    b = pl.program_id(0)
    ids = ids_ref[0, 0, :].reshape(BLK, 1)
    tok_c = lax.broadcasted_iota(jnp.int32, (BLK, T), 1)
    oh = (tok_c == jnp.broadcast_to(ids, (BLK, T))).astype(jnp.bfloat16)
    x = jnp.dot(oh, h_ref[pl.ds(0, T), :], preferred_element_type=jnp.float32)
    hid = jnp.dot(x.astype(jnp.bfloat16), w1_ref[0],
                  preferred_element_type=jnp.float32) + b1_ref[0]
    hid = jax.nn.gelu(hid)
    y = jnp.dot(hid.astype(jnp.bfloat16), w2_ref[0],
                preferred_element_type=jnp.float32) + b2_ref[0]
    y = y * wq_ref[0, 0, :].reshape(BLK, 1)
    tok_r = lax.broadcasted_iota(jnp.int32, (T, BLK), 0)
    oht = (tok_r == jnp.broadcast_to(ids.reshape(1, BLK), (T, BLK))).astype(jnp.bfloat16)
    contrib = jnp.dot(oht, y.astype(jnp.bfloat16), preferred_element_type=jnp.float32)

    @pl.when(b == 0)
    def _init():
        fin_ref[...] = contrib

    @pl.when(b > 0)
    def _acc():
        fin_ref[...] += contrib


def _route(x, W_mix, W_gate):
    return pl.pallas_call(
        _route_body,
        grid=(2 * N_RT,),
        in_specs=[
            pl.BlockSpec((RT_TILE, D), lambda i: (jnp.minimum(i, N_RT - 1), 0)),
            pl.BlockSpec((D, D), lambda i: (0, 0)),
            pl.BlockSpec((D, NUM_EXPERTS), lambda i: (0, 0)),
        ],
        out_specs=[
            pl.BlockSpec((RT_TILE, D), lambda i: (jnp.minimum(i, N_RT), 0)),
            pl.BlockSpec((RT_TILE, NUM_EXPERTS), lambda i: (jnp.minimum(i, N_RT), 0)),
            pl.BlockSpec((RT_TILE, NUM_EXPERTS), lambda i: (jnp.minimum(i, N_RT), 0)),
            pl.BlockSpec((1, 1), lambda i: (0, 0)),
            pl.BlockSpec((1, 1), lambda i: (0, 0)),
            pl.BlockSpec((1, NUM_EXPERTS), lambda i: (0, 0)),
            pl.BlockSpec((1, NUM_EXPERTS), lambda i: (0, 0)),
            pl.BlockSpec((1, NBLK), lambda i: (0, 0)),
            pl.BlockSpec((2, RT_TILE), lambda i: (0, jnp.where(i < N_RT, N_RT, i - N_RT))),
            pl.BlockSpec((2, RT_TILE), lambda i: (0, jnp.minimum(i, N_RT))),
        ],
        out_shape=[
            jax.ShapeDtypeStruct((T + RT_TILE, D), jnp.bfloat16),
            jax.ShapeDtypeStruct((T + RT_TILE, NUM_EXPERTS), jnp.float32),
            jax.ShapeDtypeStruct((T + RT_TILE, NUM_EXPERTS), jnp.float32),
            jax.ShapeDtypeStruct((1, 1), jnp.float32),
            jax.ShapeDtypeStruct((1, 1), jnp.float32),
            jax.ShapeDtypeStruct((1, NUM_EXPERTS), jnp.float32),
            jax.ShapeDtypeStruct((1, NUM_EXPERTS), jnp.int32),
            jax.ShapeDtypeStruct((1, NBLK), jnp.int32),
            jax.ShapeDtypeStruct((2, T + RT_TILE), jnp.int32),
            jax.ShapeDtypeStruct((2, T + RT_TILE), jnp.float32),
        ],
        scratch_shapes=[
            pltpu.VMEM((1, NUM_EXPERTS), jnp.float32),
            pltpu.VMEM((1, NUM_EXPERTS), jnp.float32),
            pltpu.VMEM((1, NUM_EXPERTS), jnp.float32),
            pltpu.VMEM((T, NUM_EXPERTS), jnp.float32),
            pltpu.VMEM((T, NUM_EXPERTS), jnp.float32),
        ],
    )(x, W_mix, W_gate)


def _invert_slots(pos01, w01):
    mesh = plsc.VectorSubcoreMesh(core_axis_name="c", subcore_axis_name="s")
    return pl.kernel(
        _sc_scatter_body,
        out_type=[
            jax.ShapeDtypeStruct((NPAD,), jnp.int32),
            jax.ShapeDtypeStruct((NPAD,), jnp.float32),
        ],
        mesh=mesh,
        scratch_types=[
            pltpu.VMEM((T * TOP_K,), jnp.int32),
            pltpu.VMEM((T * TOP_K,), jnp.float32),
            pltpu.VMEM((NPAD,), jnp.int32),
            pltpu.VMEM((NPAD,), jnp.float32),
        ],
        compiler_params=pltpu.CompilerParams(needs_layout_passes=False),
    )(pos01, w01)


def kernel(h_t, W_mix, W_gate, W1, b1, W2, b2):
    x = h_t.reshape(T, D)
    W1b = W1.astype(jnp.bfloat16)
    W2b = W2.astype(jnp.bfloat16)

    (h_bf, logits, probs, zloss, lbl, load, cnt_i, blkexp, pos01, w01) = _route(x, W_mix, W_gate)

    ids_buf, wq_buf = _invert_slots(pos01, w01)

    b1r = b1.reshape(NUM_EXPERTS, 1, FF)
    b2r = b2.reshape(NUM_EXPERTS, 1, D)
    wqr = wq_buf.reshape(NBLK, 1, BLK)
    idsr = ids_buf.reshape(NBLK, 1, BLK)

    final = pl.pallas_call(
        _ffn_body,
        grid_spec=pltpu.PrefetchScalarGridSpec(
            num_scalar_prefetch=1,
            grid=(NBLK,),
            in_specs=[
                pl.BlockSpec((1, 1, BLK), lambda b, be: (b, 0, 0)),
                pl.BlockSpec((T + RT_TILE, D), lambda b, be: (0, 0)),
                pl.BlockSpec((1, D, FF), lambda b, be: (be[b], 0, 0)),
                pl.BlockSpec((1, 1, FF), lambda b, be: (be[b], 0, 0)),
                pl.BlockSpec((1, FF, D), lambda b, be: (be[b], 0, 0)),
                pl.BlockSpec((1, 1, D), lambda b, be: (be[b], 0, 0)),
                pl.BlockSpec((1, 1, BLK), lambda b, be: (b, 0, 0)),
            ],
            out_specs=pl.BlockSpec((T, D), lambda b, be: (0, 0)),
        ),
        out_shape=jax.ShapeDtypeStruct((T, D), jnp.float32),
    )(blkexp.reshape(NBLK), idsr, h_bf, W1b, b1r, W2b, b2r, wqr)

    final_hidden_states = final.reshape(1, T, D)
    z_loss = zloss.reshape(())
    load_balancing_loss = lbl.reshape(())
    expert_load = load.reshape(NUM_EXPERTS)
    expert_token_counts = cnt_i.reshape(NUM_EXPERTS)
    return (logits[:T], probs[:T], final_hidden_states, z_loss,
            load_balancing_loss, expert_load, expert_token_counts)
